# baseline (speedup 1.0000x reference)
import functools

import jax
import jax.numpy as jnp
from jax import lax
from jax.experimental import pallas as pl
from jax.experimental.pallas import tpu as pltpu
from jax.experimental.pallas import tpu_sc as plsc

_R = 1000
_RPAD = 1024
_WPAD = 128
_NR = 8
_ER = 8
_D = 32
_OUT = 32
_NL = 3
_NES = 64
_NNS = 32
_NEXT = 16

_NC = 2
_NS = 16
_NW = _NC * _NS
_CHR = 8
_NBUF = 2


def _table_body(esT, nsT, eiT, evT, en0T, en1T, atT, bdT, exT,
                WlT, blT, WoT, boT, prod, table_out, idx_out):
    f32 = jnp.float32

    def onehot(idx_row, k):
        ks = lax.broadcasted_iota(jnp.int32, (k, _RPAD), 0)
        return (idx_row == ks).astype(f32)

    edge_h = []
    for e in range(_ER):
        edge_h.append(jnp.dot(atT[...], onehot(esT[e:e + 1, :], _NES),
                              preferred_element_type=f32))
    node_h = []
    for n in range(_NR):
        hb = jnp.dot(bdT[...], onehot(nsT[n:n + 1, :], _NNS),
                     preferred_element_type=f32)
        hx = jnp.dot(exT[...], onehot(eiT[n:n + 1, :], _NEXT),
                     preferred_element_type=f32)
        node_h.append(hb + evT[n:n + 1, :] * hx)

    A = []
    for e in range(_ER):
        e0 = en0T[e:e + 1, :]
        e1 = en1T[e:e + 1, :]
        A.append([(e0 == n).astype(f32) + (e1 == n).astype(f32)
                  for n in range(_NR)])

    acc = jnp.zeros((_OUT, _RPAD), f32)
    for l in range(_NL):
        Wl = WlT[_D * l:_D * (l + 1), :]
        Wo = WoT[_D * l:_D * (l + 1), :]
        bl = blT[:, l:l + 1]
        bo = boT[:, l:l + 1]
        v_e = []
        for e in range(_ER):
            m = edge_h[e]
            for n in range(_NR):
                m = m + A[e][n] * node_h[n]
            v_e.append(m)
        v_n = []
        for n in range(_NR):
            m = node_h[n]
            for e in range(_ER):
                m = m + A[e][n] * edge_h[e]
            v_n.append(m)
        for v in v_e + v_n:
            acc = acc + jnp.maximum(
                jnp.dot(Wo, v, preferred_element_type=f32) + bo, 0.0)
        for e in range(_ER):
            edge_h[e] = jnp.maximum(
                jnp.dot(Wl, v_e[e], preferred_element_type=f32) + bl, 0.0)
        for n in range(_NR):
            node_h[n] = jnp.maximum(
                jnp.dot(Wl, v_n[n], preferred_element_type=f32) + bl, 0.0)

    lane = lax.broadcasted_iota(jnp.int32, (_OUT, _RPAD), 1)
    masked = jnp.where(lane < _R, acc, 0.0)
    tp = jnp.transpose(masked)
    table_out[:, 0:_OUT] = tp
    table_out[:, _OUT:_WPAD] = jnp.zeros((_RPAD, _WPAD - _OUT), f32)

    seq_l = prod.shape[1]
    idx_out[:, 0:seq_l] = prod[...] * 4
    idx_out[:, seq_l:_WPAD] = jnp.zeros(
        (prod.shape[0], _WPAD - seq_l), jnp.int32)


def _compute_table(esT, nsT, eiT, evT, en0T, en1T, atT, bdT, exT,
                   WlT, blT, WoT, boT, prod):
    return pl.pallas_call(
        _table_body,
        out_shape=(
            jax.ShapeDtypeStruct((_RPAD, _WPAD), jnp.float32),
            jax.ShapeDtypeStruct((prod.shape[0], _WPAD), jnp.int32),
        ),
    )(esT, nsT, eiT, evT, en0T, en1T, atT, bdT, exT, WlT, blT, WoT, boT, prod)


def _sc_gather(table32, idx128, b, seq_l):
    tok = b * seq_l
    bpr = b // _NW
    mesh = plsc.VectorSubcoreMesh(core_axis_name="c", subcore_axis_name="s")

    pad_l = (seq_l + 7) // 8 * 8

    @functools.partial(
        pl.kernel,
        out_type=jax.ShapeDtypeStruct((tok, _OUT), jnp.float32),
        mesh=mesh,
        compiler_params=pltpu.CompilerParams(use_tc_tiling_on_sc=False),
        scratch_types=[
            pltpu.VMEM((bpr, _WPAD), jnp.int32),
            pltpu.VMEM((bpr * pad_l, _OUT), jnp.float32),
            pltpu.SemaphoreType.DMA,
            pltpu.SemaphoreType.DMA,
        ],
    )
    def gather_k(table_hbm, idx_hbm, out_hbm, idx_v, pack_v, gsem, wsem):
        wid = lax.axis_index("s") * _NC + lax.axis_index("c")
        rbase = wid * bpr
        tbase = wid * bpr * seq_l
        pltpu.sync_copy(idx_hbm.at[pl.ds(rbase, bpr)], idx_v)
        ghs = []
        for r in range(bpr):
            ghs.append(pltpu.async_copy(
                table_hbm.at[idx_v.at[r, pl.ds(0, pad_l)]],
                pack_v.at[pl.ds(r * pad_l, pad_l)],
                gsem))
        whs = []
        for r in range(bpr):
            ghs[r].wait()
            whs.append(pltpu.async_copy(
                pack_v.at[pl.ds(r * pad_l, seq_l)],
                out_hbm.at[pl.ds(tbase + r * seq_l, seq_l)],
                wsem))
        for w in whs:
            w.wait()

    return gather_k(table32, idx128)


def kernel(prod_rule_idx_seq, atom_embed, bond_embed, ext_id_embed,
           W_l2l, b_l2l, W_l2o, b_l2o,
           rule_edge_sym, rule_node_sym, rule_ext_id, rule_ext_valid,
           rule_edge_nodes):
    b, seq_l = prod_rule_idx_seq.shape

    def padT(x):
        return jnp.pad(x, ((0, _RPAD - _R), (0, 0))).T

    esT = padT(rule_edge_sym).astype(jnp.int32)
    nsT = padT(rule_node_sym).astype(jnp.int32)
    eiT = padT(rule_ext_id).astype(jnp.int32)
    evT = padT(rule_ext_valid).astype(jnp.float32)
    en0T = padT(rule_edge_nodes[:, :, 0]).astype(jnp.int32)
    en1T = padT(rule_edge_nodes[:, :, 1]).astype(jnp.int32)

    atT = atom_embed.T
    bdT = bond_embed.T
    exT = ext_id_embed.T
    WlT = jnp.concatenate([W_l2l[i].T for i in range(_NL)], axis=0)
    WoT = jnp.concatenate([W_l2o[i].T for i in range(_NL)], axis=0)
    blT = b_l2l.T
    boT = b_l2o.T

    table128, idx128 = _compute_table(
        esT, nsT, eiT, evT, en0T, en1T, atT, bdT, exT,
        WlT, blT, WoT, boT, prod_rule_idx_seq.astype(jnp.int32))

    table32 = table128.reshape(4 * _RPAD, _OUT)
    out_flat = _sc_gather(table32, idx128, b, seq_l)
    return out_flat.reshape(b, seq_l, _OUT)


# speedup vs baseline: 1.5985x; 1.5985x over previous
import functools

import jax
import jax.numpy as jnp
from jax import lax
from jax.experimental import pallas as pl
from jax.experimental.pallas import tpu as pltpu
from jax.experimental.pallas import tpu_sc as plsc

_R = 1000
_RPAD = 1024
_WPAD = 128
_NR = 8
_ER = 8
_D = 32
_OUT = 32
_NL = 3
_NES = 64
_NNS = 32
_NEXT = 16

_NC = 2
_NS = 16
_NW = _NC * _NS
_CHR = 8
_NBUF = 2


def _table_body(esT, nsT, eiT, evT, en0T, en1T, atT, bdT, exT,
                WlT, blT, WoT, boT, prod, table_out, idx_out):
    f32 = jnp.float32

    def onehot(idx_row, k):
        ks = lax.broadcasted_iota(jnp.int32, (k, _RPAD), 0)
        return (idx_row == ks).astype(f32)

    edge_h = []
    for e in range(_ER):
        edge_h.append(jnp.dot(atT[...], onehot(esT[e:e + 1, :], _NES),
                              preferred_element_type=f32))
    node_h = []
    for n in range(_NR):
        hb = jnp.dot(bdT[...], onehot(nsT[n:n + 1, :], _NNS),
                     preferred_element_type=f32)
        hx = jnp.dot(exT[...], onehot(eiT[n:n + 1, :], _NEXT),
                     preferred_element_type=f32)
        node_h.append(hb + evT[n:n + 1, :] * hx)

    A = []
    for e in range(_ER):
        e0 = en0T[e:e + 1, :]
        e1 = en1T[e:e + 1, :]
        A.append([(e0 == n).astype(f32) + (e1 == n).astype(f32)
                  for n in range(_NR)])

    acc = jnp.zeros((_OUT, _RPAD), f32)
    for l in range(_NL):
        Wl = WlT[_D * l:_D * (l + 1), :]
        Wo = WoT[_D * l:_D * (l + 1), :]
        bl = blT[:, l:l + 1]
        bo = boT[:, l:l + 1]
        v_e = []
        for e in range(_ER):
            m = edge_h[e]
            for n in range(_NR):
                m = m + A[e][n] * node_h[n]
            v_e.append(m)
        v_n = []
        for n in range(_NR):
            m = node_h[n]
            for e in range(_ER):
                m = m + A[e][n] * edge_h[e]
            v_n.append(m)
        for v in v_e + v_n:
            acc = acc + jnp.maximum(
                jnp.dot(Wo, v, preferred_element_type=f32) + bo, 0.0)
        for e in range(_ER):
            edge_h[e] = jnp.maximum(
                jnp.dot(Wl, v_e[e], preferred_element_type=f32) + bl, 0.0)
        for n in range(_NR):
            node_h[n] = jnp.maximum(
                jnp.dot(Wl, v_n[n], preferred_element_type=f32) + bl, 0.0)

    lane = lax.broadcasted_iota(jnp.int32, (_OUT, _RPAD), 1)
    masked = jnp.where(lane < _R, acc, 0.0)
    tp = jnp.transpose(masked)
    table_out[:, 0:_OUT] = tp
    table_out[:, _OUT:_WPAD] = jnp.zeros((_RPAD, _WPAD - _OUT), f32)

    seq_l = prod.shape[1]
    idx_out[:, 0:seq_l] = prod[...] * 4
    idx_out[:, seq_l:_WPAD] = jnp.zeros(
        (prod.shape[0], _WPAD - seq_l), jnp.int32)


def _compute_table(esT, nsT, eiT, evT, en0T, en1T, atT, bdT, exT,
                   WlT, blT, WoT, boT, prod):
    return pl.pallas_call(
        _table_body,
        out_shape=(
            jax.ShapeDtypeStruct((_RPAD, _WPAD), jnp.float32),
            jax.ShapeDtypeStruct((prod.shape[0], _WPAD), jnp.int32),
        ),
    )(esT, nsT, eiT, evT, en0T, en1T, atT, bdT, exT, WlT, blT, WoT, boT, prod)


def _sc_gather(table32, idx128, b, seq_l):
    tok = b * seq_l
    bpr = b // _NW
    mesh = plsc.VectorSubcoreMesh(core_axis_name="c", subcore_axis_name="s")

    bpw = bpr * seq_l
    nlv = (seq_l + 15) // 16
    cpad = bpr * seq_l + 16 * nlv
    chunks = [(i * 128, 128) for i in range(bpw // 128)]
    if bpw % 128:
        chunks.append((bpw - bpw % 128, bpw % 128))

    @functools.partial(
        pl.kernel,
        out_type=jax.ShapeDtypeStruct((tok, _OUT), jnp.float32),
        mesh=mesh,
        compiler_params=pltpu.CompilerParams(use_tc_tiling_on_sc=False),
        scratch_types=[
            pltpu.VMEM((bpr, _WPAD), jnp.int32),
            pltpu.VMEM((cpad,), jnp.int32),
            pltpu.VMEM((bpw, _OUT), jnp.float32),
            pltpu.SemaphoreType.DMA,
        ],
    )
    def gather_k(table_hbm, idx_hbm, out_hbm, idx_v, cflat, pack_v, gsem):
        wid = lax.axis_index("s") * _NC + lax.axis_index("c")
        rbase = wid * bpr
        tbase = wid * bpw
        pltpu.sync_copy(idx_hbm.at[pl.ds(rbase, bpr)], idx_v)
        for r in range(bpr):
            for k in range(nlv):
                cflat[pl.ds(seq_l * r + 16 * k, 16)] = idx_v[r, pl.ds(16 * k, 16)]
        ghs = []
        for off, ln in chunks:
            ghs.append(pltpu.async_copy(
                table_hbm.at[cflat.at[pl.ds(off, ln)]],
                pack_v.at[pl.ds(off, ln)],
                gsem))
        for g in ghs:
            g.wait()
        pltpu.sync_copy(pack_v, out_hbm.at[pl.ds(tbase, bpw)])

    return gather_k(table32, idx128)


def kernel(prod_rule_idx_seq, atom_embed, bond_embed, ext_id_embed,
           W_l2l, b_l2l, W_l2o, b_l2o,
           rule_edge_sym, rule_node_sym, rule_ext_id, rule_ext_valid,
           rule_edge_nodes):
    b, seq_l = prod_rule_idx_seq.shape

    def padT(x):
        return jnp.pad(x, ((0, _RPAD - _R), (0, 0))).T

    esT = padT(rule_edge_sym).astype(jnp.int32)
    nsT = padT(rule_node_sym).astype(jnp.int32)
    eiT = padT(rule_ext_id).astype(jnp.int32)
    evT = padT(rule_ext_valid).astype(jnp.float32)
    en0T = padT(rule_edge_nodes[:, :, 0]).astype(jnp.int32)
    en1T = padT(rule_edge_nodes[:, :, 1]).astype(jnp.int32)

    atT = atom_embed.T
    bdT = bond_embed.T
    exT = ext_id_embed.T
    WlT = jnp.concatenate([W_l2l[i].T for i in range(_NL)], axis=0)
    WoT = jnp.concatenate([W_l2o[i].T for i in range(_NL)], axis=0)
    blT = b_l2l.T
    boT = b_l2o.T

    table128, idx128 = _compute_table(
        esT, nsT, eiT, evT, en0T, en1T, atT, bdT, exT,
        WlT, blT, WoT, boT, prod_rule_idx_seq.astype(jnp.int32))

    table32 = table128.reshape(4 * _RPAD, _OUT)
    out_flat = _sc_gather(table32, idx128, b, seq_l)
    return out_flat.reshape(b, seq_l, _OUT)
